# baseline (speedup 1.0000x reference)
import functools

import jax
import jax.numpy as jnp
from jax import lax
from jax.experimental import pallas as pl
from jax.experimental.pallas import tpu as pltpu
from jax.experimental.pallas import tpu_sc as plsc

EMB = 64
PAD = 128
BATCH = 4096
SEQ = 200
B_TOTAL = BATCH * SEQ
NUM_WORKERS = 32
B_PER_W = B_TOTAL // NUM_WORKERS
CHUNK = 400
NBUF = 2
NCHUNK = B_PER_W // CHUNK
NOUT = NCHUNK // NBUF

_mesh = plsc.VectorSubcoreMesh(core_axis_name="c", subcore_axis_name="s")

_scratch = (
    [pltpu.VMEM((B_PER_W,), jnp.int32)]
    + [pltpu.VMEM((CHUNK, PAD), jnp.float32) for _ in range(NBUF)]
    + [pltpu.SemaphoreType.DMA for _ in range(2 * NBUF)]
)


@functools.partial(
    pl.kernel,
    mesh=_mesh,
    out_type=jax.ShapeDtypeStruct((B_TOTAL, PAD), jnp.float32),
    scratch_types=_scratch,
    compiler_params=pltpu.CompilerParams(use_tc_tiling_on_sc=True),
)
def _gather_all(idx_hbm, table_hbm, out_hbm, *scr):
    idx_v = scr[0]
    rows_v = scr[1 : 1 + NBUF]
    gsem = scr[1 + NBUF : 1 + 2 * NBUF]
    wsem = scr[1 + 2 * NBUF : 1 + 3 * NBUF]

    wid = lax.axis_index("s") * 2 + lax.axis_index("c")
    base = pl.multiple_of(wid * B_PER_W, B_PER_W)

    pltpu.sync_copy(idx_hbm.at[pl.ds(base, B_PER_W)], idx_v)

    def gather_ref(i, b):
        src = table_hbm.at[idx_v.at[pl.ds(i * CHUNK, CHUNK)]]
        return pltpu.make_async_copy(src, rows_v[b], gsem[b])

    def issue_gather(i, b):
        gather_ref(i, b).start()

    def wait_gather(i, b):
        gather_ref(i, b).wait()

    def issue_write(i, b):
        pltpu.async_copy(
            rows_v[b], out_hbm.at[pl.ds(base + i * CHUNK, CHUNK)], wsem[b]
        )

    def wait_write(b):
        pltpu.make_async_copy(
            rows_v[b], out_hbm.at[pl.ds(base, CHUNK)], wsem[b]
        ).wait()

    for b in range(NBUF):
        issue_gather(b, b)

    def outer(g, _):
        first = g * NBUF
        for b in range(NBUF):
            wait_gather(first + b, b)
            issue_write(first + b, b)
        for b in range(NBUF):
            wait_write(b)
            issue_gather(first + NBUF + b, b)
        return ()

    lax.fori_loop(0, NOUT - 1, outer, ())

    first = (NOUT - 1) * NBUF
    for b in range(NBUF):
        wait_gather(first + b, b)
        issue_write(first + b, b)
    for b in range(NBUF):
        wait_write(b)


def kernel(x, table):
    x1 = x.reshape(B_TOTAL)
    tpad = jnp.pad(table, ((0, 0), (0, PAD - EMB)))
    wide = _gather_all(x1, tpad)
    return wide.reshape(BATCH, SEQ, PAD)[:, :, :EMB]


# speedup vs baseline: 1.0058x; 1.0058x over previous
import functools

import jax
import jax.numpy as jnp
from jax import lax
from jax.experimental import pallas as pl
from jax.experimental.pallas import tpu as pltpu
from jax.experimental.pallas import tpu_sc as plsc

EMB = 64
PAD = 128
BATCH = 4096
SEQ = 200
B_TOTAL = BATCH * SEQ
NUM_WORKERS = 32
B_PER_W = B_TOTAL // NUM_WORKERS
CHUNK = 200
NBUF = 4
NCHUNK = B_PER_W // CHUNK
NOUT = NCHUNK // NBUF

_mesh = plsc.VectorSubcoreMesh(core_axis_name="c", subcore_axis_name="s")

_scratch = (
    [pltpu.VMEM((B_PER_W,), jnp.int32)]
    + [pltpu.VMEM((CHUNK, PAD), jnp.float32) for _ in range(NBUF)]
    + [pltpu.SemaphoreType.DMA for _ in range(2 * NBUF)]
)


@functools.partial(
    pl.kernel,
    mesh=_mesh,
    out_type=jax.ShapeDtypeStruct((B_TOTAL, PAD), jnp.float32),
    scratch_types=_scratch,
    compiler_params=pltpu.CompilerParams(use_tc_tiling_on_sc=True),
)
def _gather_all(idx_hbm, table_hbm, out_hbm, *scr):
    idx_v = scr[0]
    rows_v = scr[1 : 1 + NBUF]
    gsem = scr[1 + NBUF : 1 + 2 * NBUF]
    wsem = scr[1 + 2 * NBUF : 1 + 3 * NBUF]

    wid = lax.axis_index("s") * 2 + lax.axis_index("c")
    base = pl.multiple_of(wid * B_PER_W, B_PER_W)

    pltpu.sync_copy(idx_hbm.at[pl.ds(base, B_PER_W)], idx_v)

    def gather_ref(i, b):
        src = table_hbm.at[idx_v.at[pl.ds(i * CHUNK, CHUNK)]]
        return pltpu.make_async_copy(src, rows_v[b], gsem[b])

    def issue_gather(i, b):
        gather_ref(i, b).start()

    def wait_gather(i, b):
        gather_ref(i, b).wait()

    def issue_write(i, b):
        pltpu.async_copy(
            rows_v[b], out_hbm.at[pl.ds(base + i * CHUNK, CHUNK)], wsem[b]
        )

    def wait_write(b):
        pltpu.make_async_copy(
            rows_v[b], out_hbm.at[pl.ds(base, CHUNK)], wsem[b]
        ).wait()

    for b in range(NBUF):
        issue_gather(b, b)

    def outer(g, _):
        first = g * NBUF
        for b in range(NBUF):
            wait_gather(first + b, b)
            issue_write(first + b, b)
        for b in range(NBUF):
            wait_write(b)
            issue_gather(first + NBUF + b, b)
        return ()

    lax.fori_loop(0, NOUT - 1, outer, ())

    first = (NOUT - 1) * NBUF
    for b in range(NBUF):
        wait_gather(first + b, b)
        issue_write(first + b, b)
    for b in range(NBUF):
        wait_write(b)


def kernel(x, table):
    x1 = x.reshape(B_TOTAL)
    tpad = jnp.pad(table, ((0, 0), (0, PAD - EMB)))
    wide = _gather_all(x1, tpad)
    return wide.reshape(BATCH, SEQ, PAD)[:, :, :EMB]
